# baseline (speedup 1.0000x reference)
import functools

import jax
import jax.numpy as jnp
from jax import lax
from jax.experimental import pallas as pl
from jax.experimental.pallas import tpu as pltpu
from jax.experimental.pallas import tpu_sc as plsc

EMBED = 64
NF = 5
BATCH = 16384
OUT_W = NF * EMBED

_info = plsc.get_sparse_core_info()
_NW = _info.num_cores * _info.num_subcores
_BPW = BATCH // _NW
_CH = 64
_NCH = _BPW // _CH
_G = _CH // 16


def kernel(user_id, episodes, popularity, year, studio,
           user_table, episodes_table, popularity_table, year_table, studio_table):

    @functools.partial(
        pl.kernel,
        mesh=plsc.VectorSubcoreMesh(core_axis_name="c", subcore_axis_name="s"),
        out_type=jax.ShapeDtypeStruct((BATCH, OUT_W), jnp.float32),
        scratch_types=[
            [pltpu.VMEM((_BPW,), jnp.int32) for _ in range(NF)],
            [pltpu.VMEM((_CH, EMBED), jnp.float32) for _ in range(NF)],
            pltpu.VMEM((_CH, OUT_W), jnp.float32),
            pltpu.SemaphoreType.DMA,
        ],
        compiler_params=pltpu.CompilerParams(use_tc_tiling_on_sc=True),
    )
    def run(uid, ep, pop, yr, st, ut, et, pt, yt, stt, out,
            idx_v, rows_v, stage_v, sem):
        wid = lax.axis_index("s") * _info.num_cores + lax.axis_index("c")
        base = wid * _BPW
        idx_hbm = [uid, ep, pop, yr, st]
        tables = [ut, et, pt, yt, stt]

        for t in range(NF):
            pltpu.sync_copy(idx_hbm[t].at[pl.ds(base, _BPW)], idx_v[t])

        def chunk(c, _):
            for t in range(NF):
                def issue(g, _, _t=t):
                    v = idx_v[_t][pl.ds(c * _CH + g * 16, 16)]
                    for lane in range(16):
                        pltpu.async_copy(
                            tables[_t].at[pl.ds(v[lane], 1), :],
                            rows_v[_t].at[pl.ds(g * 16 + lane, 1), :],
                            sem)
                    return 0
                lax.fori_loop(0, _G, issue, 0)

            def drain(g, _):
                for lane in range(16):
                    pltpu.make_async_copy(
                        tables[0].at[pl.ds(0, 1), :],
                        rows_v[0].at[pl.ds(0, 1), :],
                        sem).wait()
                return 0
            lax.fori_loop(0, NF * _G, drain, 0)

            def asm(i, _):
                for t in range(NF):
                    for g in range(EMBED // 16):
                        stage_v[i, pl.ds(t * EMBED + g * 16, 16)] = (
                            rows_v[t][i, pl.ds(g * 16, 16)])
                return 0
            lax.fori_loop(0, _CH, asm, 0)

            pltpu.sync_copy(stage_v, out.at[pl.ds(base + c * _CH, _CH), :])
            return 0

        lax.fori_loop(0, _NCH, chunk, 0)

    return run(user_id, episodes, popularity, year, studio,
               user_table, episodes_table, popularity_table, year_table,
               studio_table)
